# baseline (speedup 1.0000x reference)
import functools

import jax
import jax.numpy as jnp
from jax import lax
from jax.experimental import pallas as pl
from jax.experimental.pallas import tpu as pltpu
from jax.experimental.pallas import tpu_sc as plsc

_NBUF = 5
_BB = 128


@functools.lru_cache(maxsize=None)
def _make_sc_kernel(batch: int, seq_len: int, d: int):
    info = plsc.get_sparse_core_info()
    nc, ns = info.num_cores, info.num_subcores
    nw = nc * ns
    n_bt = batch // _BB
    sw = nw // n_bt
    ns_per_w = seq_len // sw

    mesh = plsc.VectorSubcoreMesh(core_axis_name="c", subcore_axis_name="s")

    @functools.partial(
        pl.kernel,
        mesh=mesh,
        compiler_params=pltpu.CompilerParams(
            use_tc_tiling_on_sc=False, needs_layout_passes=False
        ),
        out_type=jax.ShapeDtypeStruct((seq_len * d, batch), jnp.float32),
        scratch_types=[
            pltpu.VMEM((ns_per_w, _BB), jnp.int32),
            pltpu.VMEM((ns_per_w, d), jnp.float32),
            [pltpu.VMEM((_BB, d), jnp.float32)] * _NBUF,
            [pltpu.VMEM((d, _BB), jnp.float32)] * _NBUF,
            [pltpu.SemaphoreType.DMA] * _NBUF,
            [pltpu.SemaphoreType.DMA] * _NBUF,
        ],
    )
    def k(xt_hbm, tok_hbm, pos_hbm, out_hbm, idx_v, pos_v, gbufs, tbufs,
          gsems, ssems):
        wid = lax.axis_index("s") * nc + lax.axis_index("c")
        bt = wid // sw
        s0 = (wid - bt * sw) * ns_per_w
        pltpu.sync_copy(
            xt_hbm.at[pl.ds(s0, ns_per_w), pl.ds(bt * _BB, _BB)], idx_v
        )
        pltpu.sync_copy(pos_hbm.at[pl.ds(s0, ns_per_w)], pos_v)

        def gather_start(g, slot):
            pltpu.async_copy(
                tok_hbm.at[idx_v.at[g]], gbufs[slot], gsems[slot]
            )

        def gather_wait(g, slot):
            pltpu.make_async_copy(
                tok_hbm.at[idx_v.at[g]], gbufs[slot], gsems[slot]
            ).wait()

        def out_slab(g):
            return out_hbm.at[
                pl.ds((s0 + g) * d, d), pl.ds(bt * _BB, _BB)
            ]

        def store_start(g, slot):
            pltpu.async_copy(tbufs[slot], out_slab(g), ssems[slot])

        def store_wait(g, slot):
            pltpu.make_async_copy(tbufs[slot], out_slab(g), ssems[slot]).wait()

        iota = lax.iota(jnp.int32, 16)

        def transpose_add(g, slot):
            gbuf = gbufs[slot]
            tbuf = tbufs[slot]

            grow = jnp.full((16,), g, jnp.int32)

            def col_body(dd, carry):
                cols = jnp.full((16,), dd, jnp.int32)
                p = plsc.load_gather(pos_v, [grow, cols])
                for b0 in range(_BB // 16):
                    rows = iota + (16 * b0)
                    v = plsc.load_gather(gbuf, [rows, cols])
                    tbuf[dd, pl.ds(16 * b0, 16)] = v + p
                return carry

            lax.fori_loop(0, d, col_body, 0)

        gather_start(0, 0)
        gather_start(1, 1)

        def outer(oi, carry):
            for b in range(_NBUF):
                g = oi * _NBUF + b

                @pl.when(g + 2 < ns_per_w)
                def _():
                    gather_start(g + 2, (b + 2) % _NBUF)

                gather_wait(g, b)

                @pl.when(g - 4 >= 0)
                def _():
                    store_wait(g - 4, (b + 1) % _NBUF)

                transpose_add(g, b)
                store_start(g, b)
            return carry

        lax.fori_loop(0, ns_per_w // _NBUF, outer, 0)

        for g in range(ns_per_w - 4, ns_per_w):
            store_wait(g, g % _NBUF)

    return k


def kernel(x, token_table, pos_table):
    b, s = x.shape
    d = token_table.shape[1]
    xt = x.astype(jnp.int32).T
    out = _make_sc_kernel(b, s, d)(xt, token_table, pos_table)
    return out.reshape(s, d, b).transpose(2, 0, 1)


# speedup vs baseline: 1.4883x; 1.4883x over previous
import functools

import jax
import jax.numpy as jnp
from jax import lax
from jax.experimental import pallas as pl
from jax.experimental.pallas import tpu as pltpu
from jax.experimental.pallas import tpu_sc as plsc

_NBUF = 5
_BB = 128


@functools.lru_cache(maxsize=None)
def _make_sc_kernel(batch: int, seq_len: int, d: int):
    info = plsc.get_sparse_core_info()
    nc, ns = info.num_cores, info.num_subcores
    nw = nc * ns
    n_bt = batch // _BB
    sw = nw // n_bt
    ns_per_w = seq_len // sw

    mesh = plsc.VectorSubcoreMesh(core_axis_name="c", subcore_axis_name="s")

    @functools.partial(
        pl.kernel,
        mesh=mesh,
        compiler_params=pltpu.CompilerParams(
            use_tc_tiling_on_sc=False, needs_layout_passes=False
        ),
        out_type=jax.ShapeDtypeStruct((seq_len * d, batch), jnp.float32),
        scratch_types=[
            pltpu.VMEM((ns_per_w, _BB), jnp.int32),
            pltpu.VMEM((ns_per_w, d), jnp.float32),
            [pltpu.VMEM((_BB, d), jnp.float32)] * _NBUF,
            [pltpu.VMEM((d, _BB), jnp.float32)] * _NBUF,
            [pltpu.SemaphoreType.DMA] * _NBUF,
            [pltpu.SemaphoreType.DMA] * _NBUF,
        ],
    )
    def k(xt_hbm, tok_hbm, pos_hbm, out_hbm, idx_v, pos_v, gbufs, tbufs,
          gsems, ssems):
        wid = lax.axis_index("s") * nc + lax.axis_index("c")
        bt = wid // sw
        s0 = (wid - bt * sw) * ns_per_w
        pltpu.sync_copy(
            xt_hbm.at[pl.ds(s0, ns_per_w), pl.ds(bt * _BB, _BB)], idx_v
        )
        pltpu.sync_copy(pos_hbm.at[pl.ds(s0, ns_per_w)], pos_v)

        def gather_start(g, slot):
            pltpu.async_copy(
                tok_hbm.at[idx_v.at[g]], gbufs[slot], gsems[slot]
            )

        def gather_wait(g, slot):
            pltpu.make_async_copy(
                tok_hbm.at[idx_v.at[g]], gbufs[slot], gsems[slot]
            ).wait()

        def out_slab(g):
            return out_hbm.at[
                pl.ds((s0 + g) * d, d), pl.ds(bt * _BB, _BB)
            ]

        def store_start(g, slot):
            pltpu.async_copy(tbufs[slot], out_slab(g), ssems[slot])

        def store_wait(g, slot):
            pltpu.make_async_copy(tbufs[slot], out_slab(g), ssems[slot]).wait()

        iota = lax.iota(jnp.int32, 16)
        rowidx = [iota + (16 * b0) for b0 in range(_BB // 16)]

        def transpose_add(g, slot):
            gbuf = gbufs[slot]
            tbuf = tbufs[slot]

            grow = jnp.full((16,), g, jnp.int32)

            @plsc.parallel_loop(0, d, step=4)
            def col_body(d0):
                for dj in range(4):
                    dd = d0 + dj
                    cols = jnp.full((16,), dd, jnp.int32)
                    p = plsc.load_gather(pos_v, [grow, cols])
                    for b0 in range(_BB // 16):
                        v = plsc.load_gather(gbuf, [rowidx[b0], cols])
                        tbuf[dd, pl.ds(16 * b0, 16)] = v + p

        gather_start(0, 0)
        gather_start(1, 1)

        def outer(oi, carry):
            for b in range(_NBUF):
                g = oi * _NBUF + b

                @pl.when(g + 2 < ns_per_w)
                def _():
                    gather_start(g + 2, (b + 2) % _NBUF)

                gather_wait(g, b)

                @pl.when(g - 4 >= 0)
                def _():
                    store_wait(g - 4, (b + 1) % _NBUF)

                transpose_add(g, b)
                store_start(g, b)
            return carry

        lax.fori_loop(0, ns_per_w // _NBUF, outer, 0)

        for g in range(ns_per_w - 4, ns_per_w):
            store_wait(g, g % _NBUF)

    return k


def kernel(x, token_table, pos_table):
    b, s = x.shape
    d = token_table.shape[1]
    xt = x.astype(jnp.int32).T
    out = _make_sc_kernel(b, s, d)(xt, token_table, pos_table)
    return out.reshape(s, d, b).transpose(2, 0, 1)
